# Initial kernel scaffold; baseline (speedup 1.0000x reference)
import jax
import jax.numpy as jnp
from jax.experimental import pallas as pl


def kernel(user_emb, item_emb, edge_index):
    raise NotImplementedError("write your pallas kernel here")


# speedup vs baseline: 7.9051x; 7.9051x over previous
import functools

import jax
import jax.numpy as jnp
from jax import lax
from jax.experimental import pallas as pl
from jax.experimental.pallas import tpu as pltpu
from jax.experimental.pallas import tpu_sc as plsc

f32 = jnp.float32
i32 = jnp.int32

_NU = 25000
_NN = 50000
_D = 64
_E = 800000
_NC = 2
_NS = 16
_L = 16
_HALF = _NN // _NC
_TRASH = _HALF
_ACC_ROWS = _HALF + 8
_EPT = _E // _NS
_T = 1600

_mesh = plsc.VectorSubcoreMesh(core_axis_name="c", subcore_axis_name="s")
_cparams = pltpu.CompilerParams(needs_layout_passes=False,
                                use_tc_tiling_on_sc=False)


def _localize(rv, base):
  loc = rv - base
  ok = (loc >= 0) & (loc < _HALF)
  return jnp.where(ok, loc, _TRASH)


def _rsqrt16(dv):
  ii = plsc.bitcast(dv, i32)
  ii = jnp.full((_L,), 0x5F3759DF, i32) - lax.shift_right_arithmetic(ii, 1)
  y = plsc.bitcast(ii, f32)
  for _ in range(3):
    y = y * (1.5 - 0.5 * dv * y * y)
  return jnp.where(dv > 0.0, y, 0.0)


def _zero_rows(buf, n):
  w = buf.shape[1]

  def body(i, _):
    for q in range(w // _L):
      buf[i, pl.ds(q * _L, _L)] = jnp.zeros((_L,), f32)
    return 0

  lax.fori_loop(0, n, body, 0)


def _zero_table(src, table, s, rows):
  z0 = jnp.minimum(s * _T, _HALF - _T)

  def zloop(k, _):
    pltpu.sync_copy(src, table.at[pl.ds(z0 + k * rows, rows), :])
    return 0

  lax.fori_loop(0, _T // rows, zloop, 0)

  @pl.when(s == 0)
  def _():
    pltpu.sync_copy(src.at[pl.ds(0, 8), :], table.at[pl.ds(_HALF, 8), :])


def _deg_body(rows_h, x0_h, dinv_h, g0_h,
              ones_v, rbuf, ibuf, dbuf, dvbuf, xbuf, deg_sp, sem):
  C, NSUB = 1024, 8
  NFULL = _EPT // C
  TAIL = _EPT - NFULL * C
  TAILG = TAIL // _L
  R = 160

  c = lax.axis_index("c")
  s = lax.axis_index("s")
  base = c * _HALF

  def ones_fill(i, _):
    ones_v[i, :] = jnp.full((_L,), 1.0, f32)
    return 0

  lax.fori_loop(0, C, ones_fill, 0)
  _zero_rows(dvbuf, R)
  _zero_table(dvbuf, deg_sp, s, R)
  plsc.subcore_barrier()

  e0 = s * _EPT

  def scatter_chunk():
    descs = [
        pltpu.async_copy(ones_v.at[pl.ds(q * 128, 128), :],
                         deg_sp.at[ibuf.at[q]], sem, add=True)
        for q in range(NSUB)
    ]
    for d in descs:
      d.wait()

  def chunk(j, _):
    pltpu.sync_copy(rows_h.at[pl.ds(e0 + j * C, C)], rbuf)
    for g in range(C // _L):
      rv = rbuf[pl.ds(g * _L, _L)]
      ibuf[g // 8, pl.ds((g % 8) * _L, _L)] = _localize(rv, base)
    scatter_chunk()
    return 0

  lax.fori_loop(0, NFULL, chunk, 0)
  pltpu.sync_copy(rows_h.at[pl.ds(e0 + NFULL * C, TAIL)],
                  rbuf.at[pl.ds(0, TAIL)])
  for g in range(TAILG):
    rv = rbuf[pl.ds(g * _L, _L)]
    ibuf[g // 8, pl.ds((g % 8) * _L, _L)] = _localize(rv, base)
  for g in range(TAILG, C // _L):
    ibuf[g // 8, pl.ds((g % 8) * _L, _L)] = jnp.full((_L,), _TRASH, i32)
  scatter_chunk()
  plsc.subcore_barrier()

  r0 = jnp.minimum(s * _T, _HALF - _T)

  def dchunk(k, _):
    row0 = r0 + k * R
    pltpu.sync_copy(deg_sp.at[pl.ds(row0, R), :], dbuf)
    pltpu.sync_copy(x0_h.at[pl.ds(base + row0, R), :], xbuf)
    for i in range(R):
      dvv = _rsqrt16(dbuf[i, :])
      dvbuf[i, :] = dvv
      for q in range(_D // _L):
        xbuf[i, pl.ds(q * _L, _L)] = xbuf[i, pl.ds(q * _L, _L)] * dvv
    pltpu.sync_copy(dvbuf, dinv_h.at[pl.ds(base + row0, R), :])
    pltpu.sync_copy(xbuf, g0_h.at[pl.ds(base + row0, R), :])
    return 0

  lax.fori_loop(0, _T // R, dchunk, 0)


def _layer_body(last, cols_h, rows_h, dinv_h, g_h, sum_h, *refs):
  if last:
    (out_h, cbuf, rbuf, ibuf, gbuf, abuf, dvbuf, sbuf, acc_sp,
     semg, sems) = refs
    go_h = None
  else:
    (out_h, go_h, cbuf, rbuf, ibuf, gbuf, abuf, dvbuf, sbuf, acc_sp,
     semg, sems) = refs
  C, NSUB = 256, 2
  NFULL = _EPT // C
  TAIL = _EPT - NFULL * C
  TAILG = TAIL // _L
  R = 64

  c = lax.axis_index("c")
  s = lax.axis_index("s")
  base = c * _HALF

  _zero_rows(sbuf, R)
  _zero_table(sbuf, acc_sp, s, R)
  plsc.subcore_barrier()

  e0 = s * _EPT

  def gather_chunk():
    descs = [
        pltpu.async_copy(g_h.at[cbuf.at[pl.ds(q * 128, 128)]],
                         gbuf.at[pl.ds(q * 128, 128), :], semg)
        for q in range(NSUB)
    ]
    for d in descs:
      d.wait()

  def scatter_chunk():
    descs = [
        pltpu.async_copy(gbuf.at[pl.ds(q * 128, 128), :],
                         acc_sp.at[ibuf.at[q]], sems, add=True)
        for q in range(NSUB)
    ]
    for d in descs:
      d.wait()

  def chunk(j, _):
    pltpu.sync_copy(cols_h.at[pl.ds(e0 + j * C, C)], cbuf)
    pltpu.sync_copy(rows_h.at[pl.ds(e0 + j * C, C)], rbuf)
    gather_chunk()
    for g in range(C // _L):
      rv = rbuf[pl.ds(g * _L, _L)]
      ibuf[g // 8, pl.ds((g % 8) * _L, _L)] = _localize(rv, base)
    scatter_chunk()
    return 0

  lax.fori_loop(0, NFULL, chunk, 0)
  pltpu.sync_copy(cols_h.at[pl.ds(e0 + NFULL * C, TAIL)],
                  cbuf.at[pl.ds(0, TAIL)])
  pltpu.sync_copy(rows_h.at[pl.ds(e0 + NFULL * C, TAIL)],
                  rbuf.at[pl.ds(0, TAIL)])
  for g in range(TAILG, C // _L):
    cbuf[pl.ds(g * _L, _L)] = jnp.zeros((_L,), i32)
  gather_chunk()
  for g in range(TAILG):
    rv = rbuf[pl.ds(g * _L, _L)]
    ibuf[g // 8, pl.ds((g % 8) * _L, _L)] = _localize(rv, base)
  for g in range(TAILG, C // _L):
    ibuf[g // 8, pl.ds((g % 8) * _L, _L)] = jnp.full((_L,), _TRASH, i32)
  scatter_chunk()
  plsc.subcore_barrier()

  r0 = jnp.minimum(s * _T, _HALF - _T)

  def dchunk(k, _):
    row0 = r0 + k * R
    pltpu.sync_copy(acc_sp.at[pl.ds(row0, R), :], abuf)
    pltpu.sync_copy(sum_h.at[pl.ds(base + row0, R), :], sbuf)
    pltpu.sync_copy(dinv_h.at[pl.ds(base + row0, R), :], dvbuf)
    for i in range(R):
      dvv = dvbuf[i, :]
      for q in range(_D // _L):
        xv = abuf[i, pl.ds(q * _L, _L)] * dvv
        sv = sbuf[i, pl.ds(q * _L, _L)] + xv
        if last:
          sbuf[i, pl.ds(q * _L, _L)] = sv * 0.25
        else:
          sbuf[i, pl.ds(q * _L, _L)] = sv
          abuf[i, pl.ds(q * _L, _L)] = xv * dvv
    pltpu.sync_copy(sbuf, out_h.at[pl.ds(base + row0, R), :])
    if not last:
      pltpu.sync_copy(abuf, go_h.at[pl.ds(base + row0, R), :])
    return 0

  lax.fori_loop(0, _T // R, dchunk, 0)


def _make_deg():
  return pl.kernel(
      _deg_body,
      out_type=(jax.ShapeDtypeStruct((_NN, _L), f32),
                jax.ShapeDtypeStruct((_NN, _D), f32)),
      mesh=_mesh,
      compiler_params=_cparams,
      scratch_types=[
          pltpu.VMEM((1024, _L), f32),
          pltpu.VMEM((1024,), i32),
          pltpu.VMEM((8, 128), i32),
          pltpu.VMEM((160, _L), f32),
          pltpu.VMEM((160, _L), f32),
          pltpu.VMEM((160, _D), f32),
          pltpu.VMEM_SHARED((_ACC_ROWS, _L), f32),
          pltpu.SemaphoreType.DMA,
      ],
  )


def _make_layer(last):
  if last:
    outs = jax.ShapeDtypeStruct((_NN, _D), f32)
  else:
    outs = (jax.ShapeDtypeStruct((_NN, _D), f32),
            jax.ShapeDtypeStruct((_NN, _D), f32))
  return pl.kernel(
      functools.partial(_layer_body, last),
      out_type=outs,
      mesh=_mesh,
      compiler_params=_cparams,
      scratch_types=[
          pltpu.VMEM((256,), i32),
          pltpu.VMEM((256,), i32),
          pltpu.VMEM((2, 128), i32),
          pltpu.VMEM((256, _D), f32),
          pltpu.VMEM((64, _D), f32),
          pltpu.VMEM((64, _L), f32),
          pltpu.VMEM((64, _D), f32),
          pltpu.VMEM_SHARED((_ACC_ROWS, _D), f32),
          pltpu.SemaphoreType.DMA,
          pltpu.SemaphoreType.DMA,
      ],
  )


_deg_kernel = _make_deg()
_layer_kernel = _make_layer(False)
_layer_kernel_last = _make_layer(True)


def kernel(user_emb, item_emb, edge_index):
  x0 = jnp.concatenate([user_emb, item_emb], axis=0)
  rows = edge_index[0]
  cols = edge_index[1]
  dinv, g0 = _deg_kernel(rows, x0)
  s1, g1 = _layer_kernel(cols, rows, dinv, g0, x0)
  s2, g2 = _layer_kernel(cols, rows, dinv, g1, s1)
  out = _layer_kernel_last(cols, rows, dinv, g2, s2)
  return (out[:_NU], out[_NU:])
